# baseline (speedup 1.0000x reference)
import functools

import jax
import jax.numpy as jnp
from jax import lax
from jax.experimental import pallas as pl
from jax.experimental.pallas import tpu as pltpu
from jax.experimental.pallas import tpu_sc as plsc

NUM_ROWS = 16384
NUM_COLS = 26
DIM = 64
B = NUM_ROWS * NUM_COLS

_info = plsc.get_sparse_core_info()
NC = _info.num_cores
NS = _info.num_subcores
NW = NC * NS
B_PER_W = B // NW
CHUNK = 512
N_CHUNKS = B_PER_W // CHUNK
NBUF = 2

assert B_PER_W * NW == B
assert N_CHUNKS * CHUNK == B_PER_W


def _body(x_hbm, w_hbm, out_hbm, idx_v, rows_v, gsem):
    wid = lax.axis_index("s") * NC + lax.axis_index("c")
    base = wid * B_PER_W

    def load_and_fire(g, b):
        pltpu.sync_copy(x_hbm.at[pl.ds(base + g * CHUNK, CHUNK)], idx_v.at[b])
        pltpu.async_copy(w_hbm.at[idx_v.at[b]], rows_v.at[b], gsem)

    def wait_and_store(g, b):
        pltpu.make_async_copy(w_hbm.at[idx_v.at[b]], rows_v.at[b], gsem).wait()
        pltpu.sync_copy(rows_v.at[b], out_hbm.at[pl.ds(base + g * CHUNK, CHUNK)])

    for b in range(NBUF):
        load_and_fire(b, b)

    def steady(gg, _):
        for b in range(NBUF):
            g = gg + b
            wait_and_store(g, b)
            load_and_fire(g + NBUF, b)
        return ()

    lax.fori_loop(0, (N_CHUNKS - NBUF) // NBUF, lambda i, c: steady(i * NBUF, c), ())

    for b in range(NBUF):
        wait_and_store(N_CHUNKS - NBUF + b, b)


def kernel(x, weight):
    mesh = plsc.VectorSubcoreMesh(core_axis_name="c", subcore_axis_name="s")
    run = functools.partial(
        pl.kernel,
        mesh=mesh,
        out_type=jax.ShapeDtypeStruct((B, DIM), jnp.float32),
        scratch_types=[
            pltpu.VMEM((NBUF, CHUNK), jnp.int32),
            pltpu.VMEM((NBUF, CHUNK, DIM), jnp.float32),
            pltpu.SemaphoreType.DMA,
        ],
        compiler_params=pltpu.CompilerParams(use_tc_tiling_on_sc=False),
    )(_body)
    out = run(x.reshape(B), weight)
    return out.reshape(NUM_ROWS, NUM_COLS, DIM)


# speedup vs baseline: 1.0055x; 1.0055x over previous
import functools

import jax
import jax.numpy as jnp
from jax import lax
from jax.experimental import pallas as pl
from jax.experimental.pallas import tpu as pltpu
from jax.experimental.pallas import tpu_sc as plsc

NUM_ROWS = 16384
NUM_COLS = 26
DIM = 64
B = NUM_ROWS * NUM_COLS

_info = plsc.get_sparse_core_info()
NC = _info.num_cores
NS = _info.num_subcores
NW = NC * NS
B_PER_W = B // NW
CHUNK = 512
N_CHUNKS = B_PER_W // CHUNK
NBUF = 3

assert B_PER_W * NW == B
assert N_CHUNKS * CHUNK == B_PER_W
assert N_CHUNKS % NBUF != 0 or True


def _body(x_hbm, w_hbm, out_hbm, idx_v, rows_v, *sems):
    gsem = sems[:NBUF]
    osem = sems[NBUF:]
    wid = lax.axis_index("s") * NC + lax.axis_index("c")
    base = wid * B_PER_W

    pltpu.sync_copy(x_hbm.at[pl.ds(base, B_PER_W)], idx_v)

    def fire_gather(g, b):
        pltpu.async_copy(
            w_hbm.at[idx_v.at[pl.ds(g * CHUNK, CHUNK)]], rows_v.at[b], gsem[b]
        )

    def wait_gather(b):
        pltpu.make_async_copy(
            w_hbm.at[idx_v.at[pl.ds(0, CHUNK)]], rows_v.at[b], gsem[b]
        ).wait()

    def fire_out(g, b):
        pltpu.async_copy(
            rows_v.at[b], out_hbm.at[pl.ds(base + g * CHUNK, CHUNK)], osem[b]
        )

    def wait_out(b):
        pltpu.make_async_copy(
            rows_v.at[b], out_hbm.at[pl.ds(base, CHUNK)], osem[b]
        ).wait()

    for g in range(NBUF - 1):
        fire_gather(g, g % NBUF)

    def step(g, b, first):
        wait_gather(b)
        fire_out(g, b)
        bf = (b + NBUF - 1) % NBUF
        if not first:
            wait_out(bf)
        fire_gather(g + NBUF - 1, bf)

    for g in range(NBUF):
        step(g, g % NBUF, first=(g == 0))

    n_steady_groups = (N_CHUNKS - (NBUF - 1)) // NBUF - 1

    def steady(i, _):
        g0 = (i + 1) * NBUF
        for b in range(NBUF):
            step(g0 + b, b, first=False)
        return ()

    lax.fori_loop(0, n_steady_groups, steady, ())

    tail_start = (n_steady_groups + 1) * NBUF
    for g in range(tail_start, N_CHUNKS):
        b = g % NBUF
        wait_gather(b)
        fire_out(g, b)
    for b in range(NBUF):
        wait_out(b)


def kernel(x, weight):
    mesh = plsc.VectorSubcoreMesh(core_axis_name="c", subcore_axis_name="s")
    run = functools.partial(
        pl.kernel,
        mesh=mesh,
        out_type=jax.ShapeDtypeStruct((B, DIM), jnp.float32),
        scratch_types=[
            pltpu.VMEM((B_PER_W,), jnp.int32),
            pltpu.VMEM((NBUF, CHUNK, DIM), jnp.float32),
        ]
        + [pltpu.SemaphoreType.DMA] * (2 * NBUF),
        compiler_params=pltpu.CompilerParams(use_tc_tiling_on_sc=False),
    )(_body)
    out = run(x.reshape(B), weight)
    return out.reshape(NUM_ROWS, NUM_COLS, DIM)


# speedup vs baseline: 1.0064x; 1.0009x over previous
import functools

import jax
import jax.numpy as jnp
from jax import lax
from jax.experimental import pallas as pl
from jax.experimental.pallas import tpu as pltpu
from jax.experimental.pallas import tpu_sc as plsc

NUM_ROWS = 16384
NUM_COLS = 26
DIM = 64
B = NUM_ROWS * NUM_COLS

_info = plsc.get_sparse_core_info()
NC = _info.num_cores
NS = _info.num_subcores
NW = NC * NS
ROWS_PER_W = NUM_ROWS // NW
B_PER_W = B // NW
CHUNK = 512
N_CHUNKS = B_PER_W // CHUNK
NBUF = 3
LANES = 16

assert ROWS_PER_W * NW == NUM_ROWS
assert N_CHUNKS * CHUNK == B_PER_W


def _body(xt_hbm, w_hbm, out_hbm, idx2d_v, idx_v, rows_v, *sems):
    gsem = sems[:NBUF]
    osem = sems[NBUF:]
    wid = lax.axis_index("s") * NC + lax.axis_index("c")
    base = wid * B_PER_W

    pltpu.sync_copy(xt_hbm.at[:, pl.ds(wid * ROWS_PER_W, ROWS_PER_W)], idx2d_v)

    lane_step = lax.broadcasted_iota(jnp.int32, (LANES,), 0) * NUM_COLS

    def tr_j(j, _):
        def tr_blk(blk, _):
            v = idx2d_v[j, pl.ds(blk * LANES, LANES)]
            dst = lane_step + (blk * LANES * NUM_COLS + j)
            plsc.store_scatter(idx_v, [dst], v)
            return ()

        lax.fori_loop(0, ROWS_PER_W // LANES, tr_blk, ())
        return ()

    lax.fori_loop(0, NUM_COLS, tr_j, ())

    def fire_gather(g, b):
        pltpu.async_copy(
            w_hbm.at[idx_v.at[pl.ds(g * CHUNK, CHUNK)]], rows_v.at[b], gsem[b]
        )

    def wait_gather(b):
        pltpu.make_async_copy(
            w_hbm.at[idx_v.at[pl.ds(0, CHUNK)]], rows_v.at[b], gsem[b]
        ).wait()

    def fire_out(g, b):
        pltpu.async_copy(
            rows_v.at[b], out_hbm.at[pl.ds(base + g * CHUNK, CHUNK)], osem[b]
        )

    def wait_out(b):
        pltpu.make_async_copy(
            rows_v.at[b], out_hbm.at[pl.ds(base, CHUNK)], osem[b]
        ).wait()

    for g in range(NBUF - 1):
        fire_gather(g, g % NBUF)

    def step(g, b, first):
        wait_gather(b)
        fire_out(g, b)
        bf = (b + NBUF - 1) % NBUF
        if not first:
            wait_out(bf)
        fire_gather(g + NBUF - 1, bf)

    for g in range(NBUF):
        step(g, g % NBUF, first=(g == 0))

    n_steady_groups = (N_CHUNKS - (NBUF - 1)) // NBUF - 1

    def steady(i, _):
        g0 = (i + 1) * NBUF
        for b in range(NBUF):
            step(g0 + b, b, first=False)
        return ()

    lax.fori_loop(0, n_steady_groups, steady, ())

    tail_start = (n_steady_groups + 1) * NBUF
    for g in range(tail_start, N_CHUNKS):
        b = g % NBUF
        wait_gather(b)
        fire_out(g, b)
    for b in range(NBUF):
        wait_out(b)


def kernel(x, weight):
    mesh = plsc.VectorSubcoreMesh(core_axis_name="c", subcore_axis_name="s")
    run = functools.partial(
        pl.kernel,
        mesh=mesh,
        out_type=jax.ShapeDtypeStruct((B, DIM), jnp.float32),
        scratch_types=[
            pltpu.VMEM((NUM_COLS, ROWS_PER_W), jnp.int32),
            pltpu.VMEM((B_PER_W,), jnp.int32),
            pltpu.VMEM((NBUF, CHUNK, DIM), jnp.float32),
        ]
        + [pltpu.SemaphoreType.DMA] * (2 * NBUF),
        compiler_params=pltpu.CompilerParams(
            use_tc_tiling_on_sc=False, needs_layout_passes=False
        ),
    )(_body)
    out = run(jnp.swapaxes(x, 0, 1), weight)
    return out.reshape(NUM_ROWS, NUM_COLS, DIM)
